# baseline (speedup 1.0000x reference)
import jax
import jax.numpy as jnp
from jax import lax
from jax.experimental import pallas as pl
from jax.experimental.pallas import tpu as pltpu
from jax.experimental.pallas import tpu_sc as plsc

N = 50000
E = 800000
D_IN = 64
QD = 16
N_CLS = 16
EP = 819200
NPAD = N + 8
ZC = 1000
CH = 1024
NCH = 50
NCH2 = 25

_f32 = jnp.float32
_bf16 = jnp.bfloat16


def _mesh():
    return plsc.VectorSubcoreMesh(
        core_axis_name="c", subcore_axis_name="s", num_cores=2, num_subcores=16
    )


def _params():
    return pltpu.CompilerParams(use_tc_tiling_on_sc=False)


def _sc1_body(xh0, xh1, srch, dsth, z32h, o32h,
              sh0, sh1, cah, cbh,
              idx_s0, idx_s1, idx_d0, idx_d1, rows0, rows1, zb16, wb16,
              fsh, gsem0, gsem1, ssem0, ssem1):
    cid = lax.axis_index("c")
    sid = lax.axis_index("s")
    idx_s = [idx_s0, idx_s1]
    idx_d = [idx_d0, idx_d1]
    rows = [rows0, rows1]
    gsem = [gsem0, gsem1]
    ssem = [ssem0, ssem1]
    pltpu.sync_copy(z32h, zb16)

    def phase(xh, outh):
        for k in range(4):
            c = sid + 16 * k

            @pl.when(c < NCH)
            def _():
                pltpu.sync_copy(zb16, fsh.at[pl.ds(c * ZC, ZC)])

        plsc.subcore_barrier()

        def gfire(j, b):
            base = (sid * NCH + j) * CH
            pltpu.sync_copy(srch.at[pl.ds(base, CH)], idx_s[b])
            pltpu.sync_copy(dsth.at[pl.ds(base, CH)], idx_d[b])
            pltpu.async_copy(xh.at[idx_s[b]], rows[b], gsem[b])

        def gwait(b):
            pltpu.make_async_copy(xh.at[idx_s[b]], rows[b], gsem[b]).wait()

        def sfire(b):
            pltpu.async_copy(rows[b], fsh.at[idx_d[b]], ssem[b], add=True)

        def swait(b):
            pltpu.make_async_copy(rows[b], fsh.at[idx_d[b]], ssem[b]).wait()

        gfire(0, 0)

        def pair(i, carry):
            @pl.when(i > 0)
            def _():
                swait(1)
            gfire(2 * i + 1, 1)
            gwait(0)
            sfire(0)
            swait(0)
            @pl.when(2 * i + 2 < NCH)
            def _():
                gfire(2 * i + 2, 0)
            gwait(1)
            sfire(1)
            return carry

        lax.fori_loop(0, NCH // 2, pair, 0)
        swait(1)
        plsc.subcore_barrier()

        for k in range(4):
            c = sid + 16 * k

            @pl.when(c < NCH)
            def _():
                pltpu.sync_copy(fsh.at[pl.ds(c * ZC, ZC)], wb16)
                pltpu.sync_copy(wb16, outh.at[pl.ds(c * ZC, ZC)])

        plsc.subcore_barrier()

    @pl.when(cid == 0)
    def _():
        phase(xh0, sh0)

    @pl.when(cid == 1)
    def _():
        phase(xh1, sh1)

    pltpu.sync_copy(o32h, rows[0])
    pltpu.sync_copy(o32h, rows[1])
    for k in range(4):
        c = sid + 16 * k

        @pl.when(c < NCH)
        def _():
            pltpu.sync_copy(zb16, fsh.at[pl.ds(c * ZC, ZC)])

    plsc.subcore_barrier()

    cnt_base = (cid * 16 + sid) * NCH2

    def cstage_fire(j, b):
        pltpu.sync_copy(dsth.at[pl.ds((cnt_base + j) * CH, CH)], idx_d[b])
        pltpu.async_copy(rows[b], fsh.at[idx_d[b]], ssem[b], add=True)

    def cswait(b):
        pltpu.make_async_copy(rows[b], fsh.at[idx_d[b]], ssem[b]).wait()

    cstage_fire(0, 0)
    cstage_fire(1, 1)

    def cbody(i, carry):
        for b in (0, 1):
            j = 2 * i + b

            @pl.when(j < NCH2)
            def _():
                cswait(b)
                cstage_fire(j, b)
        return carry

    lax.fori_loop(1, (NCH2 + 1) // 2, cbody, 0)
    cswait(0)
    cswait(1)
    plsc.subcore_barrier()

    def cwb(outh):
        for k in range(4):
            c = sid + 16 * k

            @pl.when(c < NCH)
            def _():
                pltpu.sync_copy(fsh.at[pl.ds(c * ZC, ZC)], wb16)
                pltpu.sync_copy(wb16, outh.at[pl.ds(c * ZC, ZC)])

    @pl.when(cid == 0)
    def _():
        cwb(cah)

    @pl.when(cid == 1)
    def _():
        cwb(cbh)


def _make_sc1():
    return pl.kernel(
        _sc1_body,
        out_type=[jax.ShapeDtypeStruct((N, 2 * QD), _bf16)] * 4,
        compiler_params=_params(),
        mesh=_mesh(),
        scratch_types=[
            pltpu.VMEM((CH,), jnp.int32),
            pltpu.VMEM((CH,), jnp.int32),
            pltpu.VMEM((CH,), jnp.int32),
            pltpu.VMEM((CH,), jnp.int32),
            pltpu.VMEM((CH, 2 * QD), _bf16),
            pltpu.VMEM((CH, 2 * QD), _bf16),
            pltpu.VMEM((ZC, 2 * QD), _bf16),
            pltpu.VMEM((ZC, 2 * QD), _bf16),
            pltpu.VMEM_SHARED((NPAD, 2 * QD), _bf16),
            pltpu.SemaphoreType.DMA,
            pltpu.SemaphoreType.DMA,
            pltpu.SemaphoreType.DMA,
            pltpu.SemaphoreType.DMA,
        ],
    )


def _sc2_body(ph, srch, dsth, z16h,
              s2ah, s2bh,
              idx_s0, idx_s1, idx_d0, idx_d1, rows0, rows1, zb16, psh,
              gsem0, gsem1, ssem0, ssem1):
    cid = lax.axis_index("c")
    sid = lax.axis_index("s")
    idx_s = [idx_s0, idx_s1]
    idx_d = [idx_d0, idx_d1]
    rows = [rows0, rows1]
    gsem = [gsem0, gsem1]
    ssem = [ssem0, ssem1]
    pltpu.sync_copy(z16h, zb16)
    for k in range(4):
        c = sid + 16 * k

        @pl.when(c < NCH)
        def _():
            pltpu.sync_copy(zb16, psh.at[pl.ds(c * ZC, ZC)])

    plsc.subcore_barrier()

    wid_base = (cid * 16 + sid) * NCH2

    def gfire(j, b):
        base = (wid_base + j) * CH
        pltpu.sync_copy(srch.at[pl.ds(base, CH)], idx_s[b])
        pltpu.sync_copy(dsth.at[pl.ds(base, CH)], idx_d[b])
        pltpu.async_copy(ph.at[idx_s[b]], rows[b], gsem[b])

    def gwait(b):
        pltpu.make_async_copy(ph.at[idx_s[b]], rows[b], gsem[b]).wait()

    def sfire(b):
        pltpu.async_copy(rows[b], psh.at[idx_d[b]], ssem[b], add=True)

    def swait(b):
        pltpu.make_async_copy(rows[b], psh.at[idx_d[b]], ssem[b]).wait()

    gfire(0, 0)

    def pair(i, carry):
        @pl.when(i > 0)
        def _():
            swait(1)
        gfire(2 * i + 1, 1)
        gwait(0)
        sfire(0)
        swait(0)
        @pl.when(2 * i + 2 < NCH2)
        def _():
            gfire(2 * i + 2, 0)
        gwait(1)
        sfire(1)
        return carry

    lax.fori_loop(0, NCH2 // 2, pair, 0)
    swait(1)
    gwait(0)
    sfire(0)
    swait(0)
    plsc.subcore_barrier()

    def wb(outh):
        for k in range(4):
            c = sid + 16 * k

            @pl.when(c < NCH)
            def _():
                pltpu.sync_copy(psh.at[pl.ds(c * ZC, ZC)], zb16)
                pltpu.sync_copy(zb16, outh.at[pl.ds(c * ZC, ZC)])

    @pl.when(cid == 0)
    def _():
        wb(s2ah)

    @pl.when(cid == 1)
    def _():
        wb(s2bh)


def _make_sc2():
    return pl.kernel(
        _sc2_body,
        out_type=[jax.ShapeDtypeStruct((N, N_CLS), _f32)] * 2,
        compiler_params=_params(),
        mesh=_mesh(),
        scratch_types=[
            pltpu.VMEM((CH,), jnp.int32),
            pltpu.VMEM((CH,), jnp.int32),
            pltpu.VMEM((CH,), jnp.int32),
            pltpu.VMEM((CH,), jnp.int32),
            pltpu.VMEM((CH, N_CLS), _f32),
            pltpu.VMEM((CH, N_CLS), _f32),
            pltpu.VMEM((ZC, N_CLS), _f32),
            pltpu.VMEM_SHARED((NPAD, N_CLS), _f32),
            pltpu.SemaphoreType.DMA,
            pltpu.SemaphoreType.DMA,
            pltpu.SemaphoreType.DMA,
            pltpu.SemaphoreType.DMA,
        ],
    )


def _dense1_body(sh0_r, sh1_r, ca_r, cb_r, x_r,
                 w1la_r, w1lb_r, w1r_r, b1_r,
                 w2l_r, w2r_r, b2_r, p_r, q_r):
    cnt = (ca_r[:, 0:1].astype(jnp.float32)
           + cb_r[:, 0:1].astype(jnp.float32))
    inv = 1.0 / jnp.maximum(cnt, 1.0)
    h = (jnp.dot(sh0_r[...].astype(jnp.float32) * inv, w1la_r[...])
         + jnp.dot(sh1_r[...].astype(jnp.float32) * inv, w1lb_r[...])
         + jnp.dot(x_r[...], w1r_r[...]) + b1_r[0:1, :])
    h = jnp.maximum(h, 0.0)
    p_r[...] = jnp.dot(h, w2l_r[...])
    q_r[...] = jnp.dot(h, w2r_r[...]) + b2_r[0:1, :]


def _dense2_body(a_r, b_r, ca_r, cb_r, q_r, o_r):
    cnt = (ca_r[:, 0:1].astype(jnp.float32)
           + cb_r[:, 0:1].astype(jnp.float32))
    inv = 1.0 / jnp.maximum(cnt, 1.0)
    o_r[...] = (a_r[...] + b_r[...]) * inv + q_r[...]


def kernel(x, edge_index, W1_l, b1, W1_r, W2_l, b2, W2_r):
    src = edge_index[0]
    dst = edge_index[1]
    pad = EP - E
    srcp = jnp.concatenate([src, jnp.zeros((pad,), jnp.int32)])
    dstp = jnp.concatenate([dst, jnp.full((pad,), N, jnp.int32)])
    xb = x.astype(_bf16)
    xh0 = xb[:, :2 * QD]
    xh1 = xb[:, 2 * QD:]
    z16 = jnp.zeros((ZC, QD), _f32)
    z32b = jnp.zeros((ZC, 2 * QD), _bf16)
    ones32b = jnp.ones((CH, 2 * QD), _bf16)

    sh0, sh1, cnta, cntb = _make_sc1()(xh0, xh1, srcp, dstp, z32b, ones32b)

    B = 2000
    grid = (N // B,)
    row_spec = lambda w: pl.BlockSpec((B, w), lambda i: (i, 0))
    full_spec = lambda a, b: pl.BlockSpec((a, b), lambda i: (0, 0))
    dense1 = pl.pallas_call(
        _dense1_body,
        grid=grid,
        in_specs=[
            row_spec(2 * QD), row_spec(2 * QD),
            row_spec(2 * QD), row_spec(2 * QD), row_spec(D_IN),
            full_spec(2 * QD, D_IN), full_spec(2 * QD, D_IN),
            full_spec(D_IN, D_IN), full_spec(8, D_IN),
            full_spec(D_IN, N_CLS), full_spec(D_IN, N_CLS), full_spec(8, N_CLS),
        ],
        out_specs=[row_spec(N_CLS), row_spec(N_CLS)],
        out_shape=[jax.ShapeDtypeStruct((N, N_CLS), _f32)] * 2,
    )
    w1la = W1_l[:, :2 * QD].T
    w1lb = W1_l[:, 2 * QD:].T
    b1t = jnp.tile(b1.reshape(1, D_IN), (8, 1))
    b2t = jnp.tile(b2.reshape(1, N_CLS), (8, 1))
    p, q = dense1(sh0, sh1, cnta, cntb, x,
                  w1la, w1lb, W1_r.T, b1t,
                  W2_l.T, W2_r.T, b2t)

    s2a, s2b = _make_sc2()(p, srcp, dstp, z16)

    dense2 = pl.pallas_call(
        _dense2_body,
        grid=grid,
        in_specs=[row_spec(N_CLS), row_spec(N_CLS), row_spec(2 * QD),
                  row_spec(2 * QD), row_spec(N_CLS)],
        out_specs=row_spec(N_CLS),
        out_shape=jax.ShapeDtypeStruct((N, N_CLS), _f32),
    )
    return dense2(s2a, s2b, cnta, cntb, q)


# speedup vs baseline: 1.0271x; 1.0271x over previous
import jax
import jax.numpy as jnp
from jax import lax
from jax.experimental import pallas as pl
from jax.experimental.pallas import tpu as pltpu
from jax.experimental.pallas import tpu_sc as plsc

N = 50000
E = 800000
D_IN = 64
QD = 16
N_CLS = 16
EP = 819200
NPAD = N + 8
ZC = 1000
CH = 1024
NCH = 50
NCH2 = 25

_f32 = jnp.float32
_bf16 = jnp.bfloat16


def _mesh():
    return plsc.VectorSubcoreMesh(
        core_axis_name="c", subcore_axis_name="s", num_cores=2, num_subcores=16
    )


def _params():
    return pltpu.CompilerParams(use_tc_tiling_on_sc=False)


def _cnt_body(dsth, z8h, ones8h, cah, cbh, idx_d0, idx_d1, zb8, onev, csh,
              ssem0, ssem1):
    cid = lax.axis_index("c")
    sid = lax.axis_index("s")
    idx_d = [idx_d0, idx_d1]
    ssem = [ssem0, ssem1]
    pltpu.sync_copy(z8h, zb8)
    pltpu.sync_copy(ones8h, onev)
    for k in range(4):
        c = sid + 16 * k

        @pl.when(c < NCH)
        def _():
            pltpu.sync_copy(zb8, csh.at[pl.ds(c * ZC, ZC)])

    plsc.subcore_barrier()

    wid_base = (cid * 16 + sid) * NCH2

    def stage_fire(j, b):
        pltpu.sync_copy(dsth.at[pl.ds((wid_base + j) * CH, CH)], idx_d[b])
        pltpu.async_copy(onev, csh.at[idx_d[b]], ssem[b], add=True)

    def swait(b):
        pltpu.make_async_copy(onev, csh.at[idx_d[b]], ssem[b]).wait()

    stage_fire(0, 0)
    stage_fire(1, 1)

    def body(i, carry):
        for b in (0, 1):
            j = 2 * i + b

            @pl.when(j < NCH2)
            def _():
                swait(b)
                stage_fire(j, b)
        return carry

    lax.fori_loop(1, (NCH2 + 1) // 2, body, 0)
    swait(0)
    swait(1)
    plsc.subcore_barrier()

    def wb(outh):
        for k in range(4):
            c = sid + 16 * k

            @pl.when(c < NCH)
            def _():
                pltpu.sync_copy(csh.at[pl.ds(c * ZC, ZC)], zb8)
                pltpu.sync_copy(zb8, outh.at[pl.ds(c * ZC, ZC)])

    @pl.when(cid == 0)
    def _():
        wb(cah)

    @pl.when(cid == 1)
    def _():
        wb(cbh)


def _make_cnt():
    return pl.kernel(
        _cnt_body,
        out_type=[jax.ShapeDtypeStruct((N, 8), _f32)] * 2,
        compiler_params=_params(),
        mesh=_mesh(),
        scratch_types=[
            pltpu.VMEM((CH,), jnp.int32),
            pltpu.VMEM((CH,), jnp.int32),
            pltpu.VMEM((ZC, 8), _f32),
            pltpu.VMEM((CH, 8), _f32),
            pltpu.VMEM_SHARED((NPAD, 8), _f32),
            pltpu.SemaphoreType.DMA,
            pltpu.SemaphoreType.DMA,
        ],
    )


def _sc1_body(xh0, xh1, srch, dsth, z32h,
              sh0, sh1,
              idx_s0, idx_s1, idx_d0, idx_d1, rows0, rows1, zb16, wb16,
              fsh, gsem0, gsem1, ssem0, ssem1):
    cid = lax.axis_index("c")
    sid = lax.axis_index("s")
    idx_s = [idx_s0, idx_s1]
    idx_d = [idx_d0, idx_d1]
    rows = [rows0, rows1]
    gsem = [gsem0, gsem1]
    ssem = [ssem0, ssem1]
    pltpu.sync_copy(z32h, zb16)

    def phase(xh, outh):
        for k in range(4):
            c = sid + 16 * k

            @pl.when(c < NCH)
            def _():
                pltpu.sync_copy(zb16, fsh.at[pl.ds(c * ZC, ZC)])

        plsc.subcore_barrier()

        def gfire(j, b):
            base = (sid * NCH + j) * CH
            pltpu.sync_copy(srch.at[pl.ds(base, CH)], idx_s[b])
            pltpu.sync_copy(dsth.at[pl.ds(base, CH)], idx_d[b])
            pltpu.async_copy(xh.at[idx_s[b]], rows[b], gsem[b])

        def gwait(b):
            pltpu.make_async_copy(xh.at[idx_s[b]], rows[b], gsem[b]).wait()

        def sfire(b):
            pltpu.async_copy(rows[b], fsh.at[idx_d[b]], ssem[b], add=True)

        def swait(b):
            pltpu.make_async_copy(rows[b], fsh.at[idx_d[b]], ssem[b]).wait()

        gfire(0, 0)

        def pair(i, carry):
            @pl.when(i > 0)
            def _():
                swait(1)
            gfire(2 * i + 1, 1)
            gwait(0)
            sfire(0)
            swait(0)
            @pl.when(2 * i + 2 < NCH)
            def _():
                gfire(2 * i + 2, 0)
            gwait(1)
            sfire(1)
            return carry

        lax.fori_loop(0, NCH // 2, pair, 0)
        swait(1)
        plsc.subcore_barrier()

        for k in range(4):
            c = sid + 16 * k

            @pl.when(c < NCH)
            def _():
                pltpu.sync_copy(fsh.at[pl.ds(c * ZC, ZC)], wb16)
                pltpu.sync_copy(wb16, outh.at[pl.ds(c * ZC, ZC)])

        plsc.subcore_barrier()

    @pl.when(cid == 0)
    def _():
        phase(xh0, sh0)

    @pl.when(cid == 1)
    def _():
        phase(xh1, sh1)


def _make_sc1():
    return pl.kernel(
        _sc1_body,
        out_type=[jax.ShapeDtypeStruct((N, 2 * QD), _bf16)] * 2,
        compiler_params=_params(),
        mesh=_mesh(),
        scratch_types=[
            pltpu.VMEM((CH,), jnp.int32),
            pltpu.VMEM((CH,), jnp.int32),
            pltpu.VMEM((CH,), jnp.int32),
            pltpu.VMEM((CH,), jnp.int32),
            pltpu.VMEM((CH, 2 * QD), _bf16),
            pltpu.VMEM((CH, 2 * QD), _bf16),
            pltpu.VMEM((ZC, 2 * QD), _bf16),
            pltpu.VMEM((ZC, 2 * QD), _bf16),
            pltpu.VMEM_SHARED((NPAD, 2 * QD), _bf16),
            pltpu.SemaphoreType.DMA,
            pltpu.SemaphoreType.DMA,
            pltpu.SemaphoreType.DMA,
            pltpu.SemaphoreType.DMA,
        ],
    )


def _sc2_body(ph, srch, dsth, z16h,
              s2ah, s2bh,
              idx_s0, idx_s1, idx_d0, idx_d1, rows0, rows1, zb16, psh,
              gsem0, gsem1, ssem0, ssem1):
    cid = lax.axis_index("c")
    sid = lax.axis_index("s")
    idx_s = [idx_s0, idx_s1]
    idx_d = [idx_d0, idx_d1]
    rows = [rows0, rows1]
    gsem = [gsem0, gsem1]
    ssem = [ssem0, ssem1]
    pltpu.sync_copy(z16h, zb16)
    for k in range(4):
        c = sid + 16 * k

        @pl.when(c < NCH)
        def _():
            pltpu.sync_copy(zb16, psh.at[pl.ds(c * ZC, ZC)])

    plsc.subcore_barrier()

    wid_base = (cid * 16 + sid) * NCH2

    def gfire(j, b):
        base = (wid_base + j) * CH
        pltpu.sync_copy(srch.at[pl.ds(base, CH)], idx_s[b])
        pltpu.sync_copy(dsth.at[pl.ds(base, CH)], idx_d[b])
        pltpu.async_copy(ph.at[idx_s[b]], rows[b], gsem[b])

    def gwait(b):
        pltpu.make_async_copy(ph.at[idx_s[b]], rows[b], gsem[b]).wait()

    def sfire(b):
        pltpu.async_copy(rows[b], psh.at[idx_d[b]], ssem[b], add=True)

    def swait(b):
        pltpu.make_async_copy(rows[b], psh.at[idx_d[b]], ssem[b]).wait()

    gfire(0, 0)

    def pair(i, carry):
        @pl.when(i > 0)
        def _():
            swait(1)
        gfire(2 * i + 1, 1)
        gwait(0)
        sfire(0)
        swait(0)
        @pl.when(2 * i + 2 < NCH2)
        def _():
            gfire(2 * i + 2, 0)
        gwait(1)
        sfire(1)
        return carry

    lax.fori_loop(0, NCH2 // 2, pair, 0)
    swait(1)
    gwait(0)
    sfire(0)
    swait(0)
    plsc.subcore_barrier()

    def wb(outh):
        for k in range(4):
            c = sid + 16 * k

            @pl.when(c < NCH)
            def _():
                pltpu.sync_copy(psh.at[pl.ds(c * ZC, ZC)], zb16)
                pltpu.sync_copy(zb16, outh.at[pl.ds(c * ZC, ZC)])

    @pl.when(cid == 0)
    def _():
        wb(s2ah)

    @pl.when(cid == 1)
    def _():
        wb(s2bh)


def _make_sc2():
    return pl.kernel(
        _sc2_body,
        out_type=[jax.ShapeDtypeStruct((N, N_CLS), _f32)] * 2,
        compiler_params=_params(),
        mesh=_mesh(),
        scratch_types=[
            pltpu.VMEM((CH,), jnp.int32),
            pltpu.VMEM((CH,), jnp.int32),
            pltpu.VMEM((CH,), jnp.int32),
            pltpu.VMEM((CH,), jnp.int32),
            pltpu.VMEM((CH, N_CLS), _f32),
            pltpu.VMEM((CH, N_CLS), _f32),
            pltpu.VMEM((ZC, N_CLS), _f32),
            pltpu.VMEM_SHARED((NPAD, N_CLS), _f32),
            pltpu.SemaphoreType.DMA,
            pltpu.SemaphoreType.DMA,
            pltpu.SemaphoreType.DMA,
            pltpu.SemaphoreType.DMA,
        ],
    )


def _dense1_body(sh0_r, sh1_r, ca_r, cb_r, x_r,
                 w1la_r, w1lb_r, w1r_r, b1_r,
                 w2l_r, w2r_r, b2_r, p_r, q_r):
    inv = 1.0 / jnp.maximum(ca_r[:, 0:1] + cb_r[:, 0:1], 1.0)
    h = (jnp.dot(sh0_r[...].astype(jnp.float32) * inv, w1la_r[...])
         + jnp.dot(sh1_r[...].astype(jnp.float32) * inv, w1lb_r[...])
         + jnp.dot(x_r[...], w1r_r[...]) + b1_r[0:1, :])
    h = jnp.maximum(h, 0.0)
    p_r[...] = jnp.dot(h, w2l_r[...])
    q_r[...] = jnp.dot(h, w2r_r[...]) + b2_r[0:1, :]


def _dense2_body(a_r, b_r, ca_r, cb_r, q_r, o_r):
    inv = 1.0 / jnp.maximum(ca_r[:, 0:1] + cb_r[:, 0:1], 1.0)
    o_r[...] = (a_r[...] + b_r[...]) * inv + q_r[...]


def kernel(x, edge_index, W1_l, b1, W1_r, W2_l, b2, W2_r):
    src = edge_index[0]
    dst = edge_index[1]
    pad = EP - E
    srcp = jnp.concatenate([src, jnp.zeros((pad,), jnp.int32)])
    dstp = jnp.concatenate([dst, jnp.full((pad,), N, jnp.int32)])
    xb = x.astype(_bf16)
    xh0 = xb[:, :2 * QD]
    xh1 = xb[:, 2 * QD:]
    z8 = jnp.zeros((ZC, 8), _f32)
    z16 = jnp.zeros((ZC, QD), _f32)
    z32b = jnp.zeros((ZC, 2 * QD), _bf16)
    ones8 = jnp.ones((CH, 8), _f32)

    cnta, cntb = _make_cnt()(dstp, z8, ones8)
    sh0, sh1 = _make_sc1()(xh0, xh1, srcp, dstp, z32b)

    B = 2000
    grid = (N // B,)
    row_spec = lambda w: pl.BlockSpec((B, w), lambda i: (i, 0))
    full_spec = lambda a, b: pl.BlockSpec((a, b), lambda i: (0, 0))
    dense1 = pl.pallas_call(
        _dense1_body,
        grid=grid,
        in_specs=[
            row_spec(2 * QD), row_spec(2 * QD),
            row_spec(8), row_spec(8), row_spec(D_IN),
            full_spec(2 * QD, D_IN), full_spec(2 * QD, D_IN),
            full_spec(D_IN, D_IN), full_spec(8, D_IN),
            full_spec(D_IN, N_CLS), full_spec(D_IN, N_CLS), full_spec(8, N_CLS),
        ],
        out_specs=[row_spec(N_CLS), row_spec(N_CLS)],
        out_shape=[jax.ShapeDtypeStruct((N, N_CLS), _f32)] * 2,
    )
    w1la = W1_l[:, :2 * QD].T
    w1lb = W1_l[:, 2 * QD:].T
    b1t = jnp.tile(b1.reshape(1, D_IN), (8, 1))
    b2t = jnp.tile(b2.reshape(1, N_CLS), (8, 1))
    p, q = dense1(sh0, sh1, cnta, cntb, x,
                  w1la, w1lb, W1_r.T, b1t,
                  W2_l.T, W2_r.T, b2t)

    s2a, s2b = _make_sc2()(p, srcp, dstp, z16)

    dense2 = pl.pallas_call(
        _dense2_body,
        grid=grid,
        in_specs=[row_spec(N_CLS), row_spec(N_CLS), row_spec(8), row_spec(8),
                  row_spec(N_CLS)],
        out_specs=row_spec(N_CLS),
        out_shape=jax.ShapeDtypeStruct((N, N_CLS), _f32),
    )
    return dense2(s2a, s2b, cnta, cntb, q)
